# baseline (speedup 1.0000x reference)
import functools

import jax
import jax.numpy as jnp
from jax import lax
from jax.experimental import pallas as pl


def _vq_body(x_ref, cbt_ref, idx_ref, embt_ref, *, K: int):
    x = x_ref[0]
    cbt = cbt_ref[0]
    B = x.shape[0]
    x_bf = x.astype(jnp.bfloat16)
    cbt_bf = cbt.astype(jnp.bfloat16)
    ab = jax.lax.dot_general(
        x_bf, cbt_bf, (((1,), (0,)), ((), ())),
        preferred_element_type=jnp.float32)
    a2 = jnp.sum(x * x, axis=1, keepdims=True)
    b2 = jnp.sum(cbt * cbt, axis=0, keepdims=True)
    scores = (a2 + b2) - 2.0 * ab
    minv = jnp.min(scores, axis=1, keepdims=True)
    kiota = lax.broadcasted_iota(jnp.int32, (B, K), 1)
    idx = jnp.min(jnp.where(scores == minv, kiota, K), axis=1)
    idx_ref[0, 0, :] = idx
    kiota_t = lax.broadcasted_iota(jnp.int32, (K, B), 0)
    onehot_t = (kiota_t == idx[None, :]).astype(jnp.bfloat16)
    embt_ref[0] = jax.lax.dot_general(
        cbt_bf, onehot_t, (((1,), (0,)), ((), ())),
        preferred_element_type=jnp.float32)


def kernel(input, codebook):
    B, T, D = input.shape
    K = codebook.shape[1]
    x_t = jnp.moveaxis(input, 1, 0)
    cbt = jnp.swapaxes(codebook, 1, 2)
    idx_t, embt = pl.pallas_call(
        functools.partial(_vq_body, K=K),
        grid=(T,),
        in_specs=[
            pl.BlockSpec((1, B, D), lambda t: (t, 0, 0)),
            pl.BlockSpec((1, D, K), lambda t: (t, 0, 0)),
        ],
        out_specs=[
            pl.BlockSpec((1, 1, B), lambda t: (t, 0, 0)),
            pl.BlockSpec((1, D, B), lambda t: (t, 0, 0)),
        ],
        out_shape=[
            jax.ShapeDtypeStruct((T, 1, B), jnp.int32),
            jax.ShapeDtypeStruct((T, D, B), jnp.float32),
        ],
    )(x_t, cbt)
    embed = jnp.transpose(embt, (2, 0, 1))
    return embed, idx_t[:, 0, :].T


# speedup vs baseline: 1.2871x; 1.2871x over previous
import functools

import jax
import jax.numpy as jnp
from jax import lax
from jax.experimental import pallas as pl

_TT = 2


def _vq_body(x_ref, cbt_ref, kiota_t_ref, idx_ref, embt_ref, *, K: int):
    kiota_t = kiota_t_ref[...]
    for i in range(_TT):
        x = x_ref[i]
        cbt = cbt_ref[i]
        x_bf = x.astype(jnp.bfloat16)
        cbt_bf = cbt.astype(jnp.bfloat16)
        ab = jax.lax.dot_general(
            x_bf, cbt_bf, (((1,), (0,)), ((), ())),
            preferred_element_type=jnp.float32)
        a2 = jnp.sum(x * x, axis=1, keepdims=True)
        b2 = jnp.sum(cbt * cbt, axis=0, keepdims=True)
        scores = (a2 + b2) - 2.0 * ab
        idx = jnp.argmin(scores, axis=1).astype(jnp.int32)
        idx_ref[i, 0, :] = idx
        onehot_t = (kiota_t == idx[None, :]).astype(jnp.bfloat16)
        embt_ref[i] = jax.lax.dot_general(
            cbt_bf, onehot_t, (((1,), (0,)), ((), ())),
            preferred_element_type=jnp.float32)


def kernel(input, codebook):
    B, T, D = input.shape
    K = codebook.shape[1]
    x_t = jnp.moveaxis(input, 1, 0)
    cbt = jnp.swapaxes(codebook, 1, 2)
    kiota_t = lax.broadcasted_iota(jnp.int32, (K, B), 0)
    idx_t, embt = pl.pallas_call(
        functools.partial(_vq_body, K=K),
        grid=(T // _TT,),
        in_specs=[
            pl.BlockSpec((_TT, B, D), lambda t: (t, 0, 0)),
            pl.BlockSpec((_TT, D, K), lambda t: (t, 0, 0)),
            pl.BlockSpec((K, B), lambda t: (0, 0)),
        ],
        out_specs=[
            pl.BlockSpec((_TT, 1, B), lambda t: (t, 0, 0)),
            pl.BlockSpec((_TT, D, B), lambda t: (t, 0, 0)),
        ],
        out_shape=[
            jax.ShapeDtypeStruct((T, 1, B), jnp.int32),
            jax.ShapeDtypeStruct((T, D, B), jnp.float32),
        ],
    )(x_t, cbt, kiota_t)
    embed = jnp.transpose(embt, (2, 0, 1))
    return embed, idx_t[:, 0, :].T


# speedup vs baseline: 1.3663x; 1.0616x over previous
import functools

import jax
import jax.numpy as jnp
from jax import lax
from jax.experimental import pallas as pl

_TT = 4


def _vq_body(x_ref, cbt_ref, kiota_t_ref, idx_ref, embt_ref, *, K: int):
    kiota_t = kiota_t_ref[...]
    for i in range(_TT):
        x = x_ref[i]
        cbt = cbt_ref[i]
        x_bf = x.astype(jnp.bfloat16)
        cbt_bf = cbt.astype(jnp.bfloat16)
        ab = jax.lax.dot_general(
            x_bf, cbt_bf, (((1,), (0,)), ((), ())),
            preferred_element_type=jnp.float32)
        a2 = jnp.sum(x * x, axis=1, keepdims=True)
        b2 = jnp.sum(cbt * cbt, axis=0, keepdims=True)
        scores = (a2 + b2) - 2.0 * ab
        idx = jnp.argmin(scores, axis=1).astype(jnp.int32)
        idx_ref[i, 0, :] = idx
        onehot_t = (kiota_t == idx[None, :]).astype(jnp.bfloat16)
        embt_ref[i] = jax.lax.dot_general(
            cbt_bf, onehot_t, (((1,), (0,)), ((), ())),
            preferred_element_type=jnp.float32)


def kernel(input, codebook):
    B, T, D = input.shape
    K = codebook.shape[1]
    x_t = jnp.moveaxis(input, 1, 0)
    cbt = jnp.swapaxes(codebook, 1, 2)
    kiota_t = lax.broadcasted_iota(jnp.int32, (K, B), 0)
    idx_t, embt = pl.pallas_call(
        functools.partial(_vq_body, K=K),
        grid=(T // _TT,),
        in_specs=[
            pl.BlockSpec((_TT, B, D), lambda t: (t, 0, 0)),
            pl.BlockSpec((_TT, D, K), lambda t: (t, 0, 0)),
            pl.BlockSpec((K, B), lambda t: (0, 0)),
        ],
        out_specs=[
            pl.BlockSpec((_TT, 1, B), lambda t: (t, 0, 0)),
            pl.BlockSpec((_TT, D, B), lambda t: (t, 0, 0)),
        ],
        out_shape=[
            jax.ShapeDtypeStruct((T, 1, B), jnp.int32),
            jax.ShapeDtypeStruct((T, D, B), jnp.float32),
        ],
    )(x_t, cbt, kiota_t)
    embed = jnp.transpose(embt, (2, 0, 1))
    return embed, idx_t[:, 0, :].T
